# Initial kernel scaffold; baseline (speedup 1.0000x reference)
import jax
import jax.numpy as jnp
from jax.experimental import pallas as pl


def kernel(inputs, params):
    raise NotImplementedError("write your pallas kernel here")


# speedup vs baseline: 1.9444x; 1.9444x over previous
import jax
import jax.numpy as jnp
import numpy as np
from jax.experimental import pallas as pl
from jax.experimental.pallas import tpu as pltpu

_C = 640
_N = 256
_B = 2
_R = _B * _N
_NB = 16
_KNN = [9, 9, 10, 10, 11, 12, 12, 13, 13, 14, 15, 15, 16, 16, 17, 18]
_DIL = [min(i // 4 + 1, 196 // 18) for i in range(_NB)]
_STEM_CFG = [(3, _C // 8, 2), (_C // 8, _C // 4, 2), (_C // 4, _C // 2, 2),
             (_C // 2, _C, 2), (_C, _C, 1)]
_INV = np.float32(1.0 / np.sqrt(1.0 + 1e-5))


def _bn(x, p):
    inv = 1.0 / jnp.sqrt(jnp.asarray(1.0 + 1e-5, jnp.float32))
    return x * (p["g"] * inv)[None, :, None, None] + p["be"][None, :, None, None]


def _conv3(x, p, stride):
    y = jax.lax.conv_general_dilated(x, p["w"], (stride, stride), ((1, 1), (1, 1)),
                                     dimension_numbers=("NCHW", "OIHW", "NCHW"))
    return y + p["b"][None, :, None, None]


def _conv1(x, p):
    return jnp.einsum('bchw,oc->bohw', x, p["w"]) + p["b"][None, :, None, None]


def _gelu(x):
    return jax.nn.gelu(x, approximate=False)


def _stem(x, stem):
    for i, (p, (_, _, s)) in enumerate(zip(stem, _STEM_CFG)):
        x = _bn(_conv3(x, p, s), p)
        if i < 4:
            x = _gelu(x)
    return x


def _fc1_body(x_ref, w_ref, s_ref, b_ref, o_ref):
    o_ref[...] = jax.lax.dot_general(
        x_ref[...], w_ref[...], (((1,), (1,)), ((), ())),
        preferred_element_type=jnp.float32) * s_ref[...] + b_ref[...]


def _fc1(x_tok, p):
    s = (p["g"] * _INV)[None, :]
    b = (p["b"] * (p["g"] * _INV) + p["be"])[None, :]
    return pl.pallas_call(
        _fc1_body,
        out_shape=jax.ShapeDtypeStruct((_R, _C), jnp.float32),
    )(x_tok, p["w"], s, b)


def _select_body(kd_ref, dist_ref, y_ref, o_ref, d_scr):
    d_scr[...] = dist_ref[...]
    o_ref[...] = jnp.full((_R, _C), -jnp.inf, jnp.float32)
    iota = jax.lax.broadcasted_iota(jnp.int32, (_R, _N), 1)
    kd = kd_ref[0, 0]
    dil = kd_ref[0, 1]

    def body(m, carry):
        dist = d_scr[...]
        mval = jnp.min(dist, axis=1, keepdims=True)
        idxm = jnp.min(jnp.where(dist == mval, iota, _N), axis=1, keepdims=True)
        onehot = iota == idxm
        d_scr[...] = jnp.where(onehot, jnp.float32(jnp.inf), dist)

        @pl.when(m % dil == 0)
        def _():
            oh = onehot.astype(jnp.float32)
            for b in range(_B):
                r = slice(_N * b, _N * (b + 1))
                g = jax.lax.dot_general(
                    oh[r], y_ref[r], (((1,), (0,)), ((), ())),
                    precision=jax.lax.Precision.HIGHEST,
                    preferred_element_type=jnp.float32)
                o_ref[r] = jnp.maximum(o_ref[r], g)

        return carry

    jax.lax.fori_loop(0, kd, body, 0)
    o_ref[...] = o_ref[...] - y_ref[...]


def _select_gather(dist, y, k, d):
    kd = jnp.array([[k * d, d]], jnp.int32)
    return pl.pallas_call(
        _select_body,
        in_specs=[pl.BlockSpec(memory_space=pltpu.SMEM),
                  pl.BlockSpec((_R, _N), None),
                  pl.BlockSpec((_R, _C), None)],
        out_specs=pl.BlockSpec((_R, _C), None),
        out_shape=jax.ShapeDtypeStruct((_R, _C), jnp.float32),
        scratch_shapes=[pltpu.VMEM((_R, _N), jnp.float32)],
    )(kd, dist, y)


def _grapher(x, p, k, d):
    B, Cc, H, W = x.shape
    N = H * W
    shortcut = x
    xf_tok = x.reshape(B, Cc, N).transpose(0, 2, 1).reshape(B * N, Cc)
    y = _fc1(xf_tok, p["fc1"])
    xf = y.reshape(B, N, Cc)
    xs = jax.lax.stop_gradient(xf)
    nrm = jnp.maximum(jnp.linalg.norm(xs, axis=-1, keepdims=True), 1e-12)
    xn = xs / nrm
    sq = jnp.sum(xn * xn, axis=-1)
    inner = jnp.einsum('bnc,bmc->bnm', xn, xn)
    dist = sq[:, :, None] - 2.0 * inner + sq[:, None, :]
    diff_max = _select_gather(dist.reshape(B * N, N), y, k, d)
    xt = xf.transpose(0, 2, 1)
    dm = diff_max.reshape(B, N, Cc).transpose(0, 2, 1)
    z = jnp.stack([xt, dm], axis=2).reshape(B, 2 * Cc, N)[:, :, :, None]
    z = _gelu(_bn(_conv1(z, p["mr"]), p["mr"]))
    z = z.reshape(B, 2 * Cc, H, W)
    z = _bn(_conv1(z, p["fc2"]), p["fc2"])
    return z + shortcut


def _ffn(x, p):
    shortcut = x
    x = _bn(_conv1(x, p["ffn1"]), p["ffn1"])
    x = _gelu(x)
    x = _bn(_conv1(x, p["ffn2"]), p["ffn2"])
    return x + shortcut


def kernel(inputs, params):
    x = _stem(inputs, params["stem"]) + params["pos_embed"]
    for i in range(_NB):
        bp = params["blocks"][i]
        x = _grapher(x, bp, _KNN[i], _DIL[i])
        x = _ffn(x, bp)
    return x
